# baseline (speedup 1.0000x reference)
import functools

import jax
import jax.numpy as jnp
from jax import lax
from jax.experimental import pallas as pl
from jax.experimental.pallas import tpu as pltpu
from jax.experimental.pallas import tpu_sc as plsc

N = 10000
E = 320000
D = 128
HD = D // 2
NH = 2
NC = 2
NS = 16
NW = NC * NS
EPW = E // NW
CH = 125
STEPS = EPW // CH
NP = N
RPT = NP // NS
ZR = 125
CW = 16

_f32 = jnp.float32


def _mm_body(x_ref, w_ref, o_ref):
    o_ref[...] = jnp.dot(x_ref[...], w_ref[...], preferred_element_type=_f32)


def _edge_transform(x, W_edge):
    bm = 2000
    return pl.pallas_call(
        _mm_body,
        grid=(N // bm,),
        in_specs=[
            pl.BlockSpec((bm, D), lambda i: (i, 0)),
            pl.BlockSpec((D, D), lambda i: (0, 0)),
        ],
        out_specs=pl.BlockSpec((bm, D), lambda i: (i, 0)),
        out_shape=jax.ShapeDtypeStruct((N, D), _f32),
    )(x, W_edge)


_sc_mesh = plsc.VectorSubcoreMesh(core_axis_name="c", subcore_axis_name="s")


@functools.partial(
    pl.kernel,
    mesh=_sc_mesh,
    compiler_params=pltpu.CompilerParams(use_tc_tiling_on_sc=False),
    out_type=(
        pltpu.HBM((NC, NP, D), _f32),
        pltpu.HBM((NC, NP, CW), _f32),
    ),
    scratch_types=(
        [pltpu.VMEM((STEPS, CH), jnp.int32)] * 2
        + [pltpu.VMEM((CH, HD), _f32)] * 6
        + [
            pltpu.VMEM((CH, CW), _f32),
            pltpu.VMEM((ZR, HD), _f32),
            pltpu.VMEM((ZR, CW), _f32),
            pltpu.VMEM_SHARED((NP, HD), _f32),
            pltpu.VMEM_SHARED((NP, CW), _f32),
        ]
        + [pltpu.SemaphoreType.DMA] * 7
    ),
)
def _sc_aggregate(y_hbm, ei_hbm, out_sum, out_cnt,
                  src_v, dst_v,
                  buf0, buf1, buf2, buf3, buf4, buf5,
                  ones_v, zb, zb_c, acc_sum, acc_cnt,
                  sem0, sem1, sem2, sem3, sem4, sem5, sem_c):
    c = lax.axis_index("c")
    s = lax.axis_index("s")
    w = c * NS + s

    @pl.loop(0, ZR)
    def _init(r):
        for j in range(HD // 16):
            zb[r, pl.ds(16 * j, 16)] = jnp.zeros((16,), _f32)
        zb_c[r, :] = jnp.zeros((16,), _f32)

    @pl.loop(0, CH)
    def _init_ones(r):
        ones_v[r, :] = jnp.ones((16,), _f32)

    pltpu.sync_copy(ei_hbm.at[0, w], src_v)
    pltpu.sync_copy(ei_hbm.at[1, w], dst_v)

    lane = lax.iota(jnp.int32, 16)

    @pl.loop(0, STEPS)
    def _dbl(r):
        for o in range(0, 112, 16):
            src_v[r, pl.ds(o, 16)] = src_v[r, pl.ds(o, 16)] * 2
        v = src_v[r, pl.ds(109, 16)]
        src_v[r, pl.ds(109, 16)] = jnp.where(lane < 3, v, v * 2)

    for h, y_h in ((0, y_hbm.at[pl.ds(0, 2 * N - 1)]),
                   (1, y_hbm.at[pl.ds(1, 2 * N - 1)])):
        for i in range(RPT // ZR):
            base = s * RPT + i * ZR
            pltpu.sync_copy(zb, acc_sum.at[pl.ds(base, ZR)])
            if h == 0:
                pltpu.sync_copy(zb_c, acc_cnt.at[pl.ds(base, ZR)])
        plsc.subcore_barrier()

        bufs = (buf0, buf1, buf2, buf3, buf4, buf5)
        sems = (sem0, sem1, sem2, sem3, sem4, sem5)
        NB = 6
        AH = 5
        for b in range(AH):
            pltpu.async_copy(y_h.at[src_v.at[b]], bufs[b], sems[b])

        def _one_step(j, b, first_round):
            b2 = (b + AH) % NB
            pltpu.make_async_copy(y_h.at[src_v.at[j]], bufs[b],
                                  sems[b]).wait()
            pltpu.async_copy(bufs[b], acc_sum.at[dst_v.at[j]], sems[b],
                             add=True)
            if h == 0:
                pltpu.async_copy(ones_v, acc_cnt.at[dst_v.at[j]], sem_c,
                                 add=True)

            def _refill():
                pltpu.async_copy(y_h.at[src_v.at[j + AH]], bufs[b2],
                                 sems[b2])

            if b < NB - AH:
                @pl.when(jnp.logical_not(first_round))
                def _():
                    pltpu.make_async_copy(
                        y_h.at[src_v.at[j]], bufs[b2], sems[b2]).wait()

                @pl.when(j + AH < STEPS)
                def _():
                    _refill()
            else:
                pltpu.make_async_copy(
                    y_h.at[src_v.at[j]], bufs[b2], sems[b2]).wait()

                @pl.when(j + AH < STEPS)
                def _():
                    _refill()

        @pl.loop(0, STEPS // NB)
        def _step(i):
            for b in range(NB):
                _one_step(NB * i + b, b, i == 0)

        for t in range(STEPS - (STEPS // NB) * NB):
            j = (STEPS // NB) * NB + t
            _one_step(j, j % NB, jnp.bool_(False))

        for j in range(STEPS - (NB - AH), STEPS):
            pltpu.make_async_copy(y_h.at[src_v.at[0]], bufs[j % NB],
                                  sems[j % NB]).wait()
        if h == 0:
            @pl.loop(0, STEPS)
            def _drain(i):
                pltpu.make_async_copy(ones_v, acc_cnt.at[dst_v.at[0]],
                                      sem_c).wait()

        plsc.subcore_barrier()

        for i in range(RPT // ZR):
            base = s * RPT + i * ZR
            pltpu.sync_copy(acc_sum.at[pl.ds(base, ZR)], zb)
            pltpu.sync_copy(zb, out_sum.at[c, pl.ds(base, ZR), pl.ds(h * HD, HD)])
            if h == 0:
                pltpu.sync_copy(acc_cnt.at[pl.ds(base, ZR)], zb_c)
                pltpu.sync_copy(zb_c, out_cnt.at[c, pl.ds(base, ZR)])
        if h == 0:
            plsc.subcore_barrier()

            @pl.loop(0, ZR)
            def _rezero(r):
                for j in range(HD // 16):
                    zb[r, pl.ds(16 * j, 16)] = jnp.zeros((16,), _f32)


def _combine_body(x_ref, w_ref, b_ref, ps_ref, pc_ref, o_ref):
    y2 = jnp.dot(x_ref[...], w_ref[...], preferred_element_type=_f32)
    y2 = y2 + b_ref[...]
    ssum = ps_ref[0] + ps_ref[1]
    deg = pc_ref[0, :, 0] + pc_ref[1, :, 0]
    deg = jnp.maximum(deg, 1.0)
    o_ref[...] = jnp.maximum(y2 + ssum / deg[:, None], 0.0)


def _combine(x, W_self, b_self, psum, pcnt):
    bm = 2000
    return pl.pallas_call(
        _combine_body,
        grid=(N // bm,),
        in_specs=[
            pl.BlockSpec((bm, D), lambda i: (i, 0)),
            pl.BlockSpec((D, D), lambda i: (0, 0)),
            pl.BlockSpec((1, D), lambda i: (0, 0)),
            pl.BlockSpec((NC, bm, D), lambda i: (0, i, 0)),
            pl.BlockSpec((NC, bm, CW), lambda i: (0, i, 0)),
        ],
        out_specs=pl.BlockSpec((bm, D), lambda i: (i, 0)),
        out_shape=jax.ShapeDtypeStruct((N, D), _f32),
    )(x, W_self, b_self.reshape(1, D), psum, pcnt)


def kernel(x, edge_index, W_edge, W_self, b_self):
    ei = edge_index.astype(jnp.int32)
    ei_aug = ei.reshape(2, NW, STEPS, CH)
    y = _edge_transform(x, W_edge)
    psum, pcnt = _sc_aggregate(y.reshape(2 * N, HD), ei_aug)
    return _combine(x, W_self, b_self, psum, pcnt)


# speedup vs baseline: 1.0013x; 1.0013x over previous
import functools

import jax
import jax.numpy as jnp
from jax import lax
from jax.experimental import pallas as pl
from jax.experimental.pallas import tpu as pltpu
from jax.experimental.pallas import tpu_sc as plsc

N = 10000
E = 320000
D = 128
HD = D // 2
NH = 2
NC = 2
NS = 16
NW = NC * NS
EPW = E // NW
CH = 125
STEPS = EPW // CH
NP = N
RPT = NP // NS
ZR = 125
CW = 16

_f32 = jnp.float32


def _mm_body(x_ref, w_ref, o_ref):
    o_ref[...] = jnp.dot(x_ref[...], w_ref[...], preferred_element_type=_f32)


def _edge_transform(x, W_edge):
    bm = 2000
    return pl.pallas_call(
        _mm_body,
        grid=(N // bm,),
        in_specs=[
            pl.BlockSpec((bm, D), lambda i: (i, 0)),
            pl.BlockSpec((D, D), lambda i: (0, 0)),
        ],
        out_specs=pl.BlockSpec((bm, D), lambda i: (i, 0)),
        out_shape=jax.ShapeDtypeStruct((N, D), _f32),
    )(x, W_edge)


_sc_mesh = plsc.VectorSubcoreMesh(core_axis_name="c", subcore_axis_name="s")


@functools.partial(
    pl.kernel,
    mesh=_sc_mesh,
    compiler_params=pltpu.CompilerParams(use_tc_tiling_on_sc=False),
    out_type=(
        pltpu.HBM((NC, NP, D), _f32),
        pltpu.HBM((NC, NP, CW), _f32),
    ),
    scratch_types=(
        [pltpu.VMEM((STEPS, CH), jnp.int32)] * 2
        + [pltpu.VMEM((CH, HD), _f32)] * 6
        + [
            pltpu.VMEM((CH, CW), _f32),
            pltpu.VMEM((ZR, HD), _f32),
            pltpu.VMEM((ZR, CW), _f32),
            pltpu.VMEM_SHARED((NP, HD), _f32),
            pltpu.VMEM_SHARED((NP, CW), _f32),
        ]
        + [pltpu.SemaphoreType.DMA] * 7
    ),
)
def _sc_aggregate(y_hbm, ei_hbm, out_sum, out_cnt,
                  src_v, dst_v,
                  buf0, buf1, buf2, buf3, buf4, buf5,
                  ones_v, zb, zb_c, acc_sum, acc_cnt,
                  sem0, sem1, sem2, sem3, sem4, sem5, sem_c):
    c = lax.axis_index("c")
    s = lax.axis_index("s")
    w = c * NS + s

    @pl.loop(0, ZR)
    def _init(r):
        for j in range(HD // 16):
            zb[r, pl.ds(16 * j, 16)] = jnp.zeros((16,), _f32)
        zb_c[r, :] = jnp.zeros((16,), _f32)

    @pl.loop(0, CH)
    def _init_ones(r):
        ones_v[r, :] = jnp.ones((16,), _f32)

    pltpu.sync_copy(ei_hbm.at[0, w], src_v)
    pltpu.sync_copy(ei_hbm.at[1, w], dst_v)

    lane = lax.iota(jnp.int32, 16)

    @pl.loop(0, STEPS)
    def _dbl(r):
        for o in range(0, 112, 16):
            src_v[r, pl.ds(o, 16)] = src_v[r, pl.ds(o, 16)] * 2
        v = src_v[r, pl.ds(109, 16)]
        src_v[r, pl.ds(109, 16)] = jnp.where(lane < 3, v, v * 2)

    for h, y_h in ((0, y_hbm.at[pl.ds(0, 2 * N - 1)]),
                   (1, y_hbm.at[pl.ds(1, 2 * N - 1)])):
        for i in range(RPT // ZR):
            base = s * RPT + i * ZR
            pltpu.sync_copy(zb, acc_sum.at[pl.ds(base, ZR)])
            if h == 0:
                pltpu.sync_copy(zb_c, acc_cnt.at[pl.ds(base, ZR)])
        plsc.subcore_barrier()

        bufs = (buf0, buf1, buf2, buf3, buf4, buf5)
        sems = (sem0, sem1, sem2, sem3, sem4, sem5)
        NB = 6
        AH = 4
        for b in range(AH):
            pltpu.async_copy(y_h.at[src_v.at[b]], bufs[b], sems[b])

        def _one_step(j, b, first_round):
            b2 = (b + AH) % NB
            pltpu.make_async_copy(y_h.at[src_v.at[j]], bufs[b],
                                  sems[b]).wait()
            pltpu.async_copy(bufs[b], acc_sum.at[dst_v.at[j]], sems[b],
                             add=True)
            if h == 0:
                pltpu.async_copy(ones_v, acc_cnt.at[dst_v.at[j]], sem_c,
                                 add=True)

            def _refill():
                pltpu.async_copy(y_h.at[src_v.at[j + AH]], bufs[b2],
                                 sems[b2])

            if b < NB - AH:
                @pl.when(jnp.logical_not(first_round))
                def _():
                    pltpu.make_async_copy(
                        y_h.at[src_v.at[j]], bufs[b2], sems[b2]).wait()

                @pl.when(j + AH < STEPS)
                def _():
                    _refill()
            else:
                pltpu.make_async_copy(
                    y_h.at[src_v.at[j]], bufs[b2], sems[b2]).wait()

                @pl.when(j + AH < STEPS)
                def _():
                    _refill()

        @pl.loop(0, STEPS // NB)
        def _step(i):
            for b in range(NB):
                _one_step(NB * i + b, b, i == 0)

        for t in range(STEPS - (STEPS // NB) * NB):
            j = (STEPS // NB) * NB + t
            _one_step(j, j % NB, jnp.bool_(False))

        for j in range(STEPS - (NB - AH), STEPS):
            pltpu.make_async_copy(y_h.at[src_v.at[0]], bufs[j % NB],
                                  sems[j % NB]).wait()
        if h == 0:
            @pl.loop(0, STEPS)
            def _drain(i):
                pltpu.make_async_copy(ones_v, acc_cnt.at[dst_v.at[0]],
                                      sem_c).wait()

        plsc.subcore_barrier()

        for i in range(RPT // ZR):
            base = s * RPT + i * ZR
            pltpu.sync_copy(acc_sum.at[pl.ds(base, ZR)], zb)
            pltpu.sync_copy(zb, out_sum.at[c, pl.ds(base, ZR), pl.ds(h * HD, HD)])
            if h == 0:
                pltpu.sync_copy(acc_cnt.at[pl.ds(base, ZR)], zb_c)
                pltpu.sync_copy(zb_c, out_cnt.at[c, pl.ds(base, ZR)])
        if h == 0:
            plsc.subcore_barrier()

            @pl.loop(0, ZR)
            def _rezero(r):
                for j in range(HD // 16):
                    zb[r, pl.ds(16 * j, 16)] = jnp.zeros((16,), _f32)


def _combine_body(x_ref, w_ref, b_ref, ps_ref, pc_ref, o_ref):
    y2 = jnp.dot(x_ref[...], w_ref[...], preferred_element_type=_f32)
    y2 = y2 + b_ref[...]
    ssum = ps_ref[0] + ps_ref[1]
    deg = pc_ref[0, :, 0] + pc_ref[1, :, 0]
    deg = jnp.maximum(deg, 1.0)
    o_ref[...] = jnp.maximum(y2 + ssum / deg[:, None], 0.0)


def _combine(x, W_self, b_self, psum, pcnt):
    bm = 2000
    return pl.pallas_call(
        _combine_body,
        grid=(N // bm,),
        in_specs=[
            pl.BlockSpec((bm, D), lambda i: (i, 0)),
            pl.BlockSpec((D, D), lambda i: (0, 0)),
            pl.BlockSpec((1, D), lambda i: (0, 0)),
            pl.BlockSpec((NC, bm, D), lambda i: (0, i, 0)),
            pl.BlockSpec((NC, bm, CW), lambda i: (0, i, 0)),
        ],
        out_specs=pl.BlockSpec((bm, D), lambda i: (i, 0)),
        out_shape=jax.ShapeDtypeStruct((N, D), _f32),
    )(x, W_self, b_self.reshape(1, D), psum, pcnt)


def kernel(x, edge_index, W_edge, W_self, b_self):
    ei = edge_index.astype(jnp.int32)
    ei_aug = ei.reshape(2, NW, STEPS, CH)
    y = _edge_transform(x, W_edge)
    psum, pcnt = _sc_aggregate(y.reshape(2 * N, HD), ei_aug)
    return _combine(x, W_self, b_self, psum, pcnt)
